# baseline (speedup 1.0000x reference)
import jax
import jax.numpy as jnp
from jax import lax
from jax.experimental import pallas as pl
from jax.experimental.pallas import tpu as pltpu
from jax.experimental.pallas import tpu_sc as plsc

B = 16384
L = 200
D = 16
LPAD = 208
NC, NS = 2, 16
NW = NC * NS
UPW = B // NW
C = 128
NBUF = 4
NCHUNK = UPW // C

N_USERS = 100000
N_ITEMS = 1000000
UB_UV = (N_USERS + 127) // 128
UV_PAD = UB_UV * 128
NB_V2 = (N_ITEMS + 127) // 128
V2_PAD = NB_V2 * 128


def _rsqrt16(x):
    i = plsc.bitcast(x, jnp.int32)
    i = jnp.int32(0x5F3759DF) - lax.shift_right_arithmetic(i, 1)
    y = plsc.bitcast(i, jnp.float32)
    half_x = x * 0.5
    for _ in range(3):
        y = y * (1.5 - half_x * y * y)
    return y


def _relayout_body(v2t_hbm, nb_hbm, v2lin_hbm,
                   tv20, tv21, lv20, lv21, nb0, nb1,
                   srv20, srv21, swv20, swv21, srnb0, srnb1):
    wid = lax.axis_index("c") * NS + lax.axis_index("s")
    iota = lax.iota(jnp.int32, 16)
    iota16 = iota * D

    tv2 = [tv20, tv21]
    lv2 = [lv20, lv21]
    nbv = [nb0, nb1]
    srv2 = [srv20, srv21]
    srnb = [srnb0, srnb1]
    swv2 = [swv20, swv21]
    n_v2 = (NB_V2 - wid + NW - 1) // NW

    def v2_fire_read(seq, b):
        ib = wid + seq * NW
        pltpu.async_copy(v2t_hbm.at[pl.ds(0, D), pl.ds(ib * 128, 128)],
                         tv2[b], srv2[b])
        pltpu.async_copy(nb_hbm.at[pl.ds(ib * 128, 128)], nbv[b], srnb[b])

    def v2_drain_read(b):
        pltpu.make_async_copy(v2t_hbm.at[pl.ds(0, D), pl.ds(0, 128)],
                              tv2[b], srv2[b]).wait()
        pltpu.make_async_copy(nb_hbm.at[pl.ds(0, 128)], nbv[b],
                              srnb[b]).wait()

    def v2_fire_write(seq, b):
        ib = wid + seq * NW
        pltpu.async_copy(lv2[b], v2lin_hbm.at[pl.ds(ib * 128 * D, 128 * D)],
                         swv2[b])

    def v2_drain_write(b):
        pltpu.make_async_copy(lv2[b], v2lin_hbm.at[pl.ds(0, 128 * D)],
                              swv2[b]).wait()

    def v2_process(b):
        for v16 in range(8):
            w16 = _rsqrt16(nbv[b][pl.ds(v16 * 16, 16)])
            for d in range(D):
                vals = tv2[b][d, pl.ds(v16 * 16, 16)] * w16
                idx = iota16 + (d + v16 * 16 * D)
                plsc.store_scatter(lv2[b], [idx], vals)

    v2_fire_read(0, 0)
    v2_fire_read(1, 1)

    def v2_pair(p):
        for b in range(2):
            seq = p * 2 + b

            @pl.when(seq < n_v2)
            def _():
                v2_drain_read(b)

                @pl.when(seq >= 2)
                def _():
                    v2_drain_write(b)

                v2_process(b)
                v2_fire_write(seq, b)

                @pl.when(seq + 2 < n_v2)
                def _():
                    v2_fire_read(seq + 2, b)

    pl.loop(0, (NB_V2 // NW + 2) // 2)(v2_pair)
    v2_drain_write(0)
    v2_drain_write(1)


def _gather_body(nodes_hbm, v2e_hbm, uvt_hbm, na_hbm, out_hbm,
                 nodes_v, hist_t, hist_v, na_v, e_buf, out_v,
                 sem_c, sem_h, sem_e0, sem_e1, sem_e2, sem_e3):
    sem_e = [sem_e0, sem_e1, sem_e2, sem_e3]
    wid = lax.axis_index("c") * NS + lax.axis_index("s")
    base = wid * UPW
    iota = lax.iota(jnp.int32, 16)
    iota200 = iota * L

    zero16 = jnp.zeros((D,), jnp.float32)
    for b in range(NBUF):
        for r in range(L, LPAD):
            e_buf[b, r, :] = zero16

    def fire(u, slot):
        pltpu.async_copy(v2e_hbm.at[hist_v.at[pl.ds(u * L, L)]],
                         e_buf.at[slot, pl.ds(0, L)], sem_e[slot])

    def drain(slot):
        pltpu.make_async_copy(v2e_hbm.at[hist_v.at[pl.ds(0, L)]],
                              e_buf.at[slot, pl.ds(0, L)], sem_e[slot]).wait()

    def compute(u, slot, rna_u):
        def cbody(cth, accs):
            a0, a1, a2, a3 = accs
            r = cth * 16
            for j in range(0, 16, 4):
                a0 = a0 + e_buf[slot, r + j, :]
                a1 = a1 + e_buf[slot, r + j + 1, :]
                a2 = a2 + e_buf[slot, r + j + 2, :]
                a3 = a3 + e_buf[slot, r + j + 3, :]
            return (a0, a1, a2, a3)

        z = jnp.zeros((D,), jnp.float32)
        a0, a1, a2, a3 = lax.fori_loop(0, LPAD // 16, cbody, (z, z, z, z))
        out_v[u, :] = ((a0 + a1) + (a2 + a3)) * rna_u

    def chunk(ch):
        ubase = base + ch * C
        pltpu.sync_copy(nodes_hbm.at[pl.ds(ubase, C)], nodes_v)
        pltpu.async_copy(na_hbm.at[nodes_v], na_v, sem_c)

        def h_fire(l):
            pltpu.async_copy(uvt_hbm.at[l].at[nodes_v], hist_t.at[l], sem_h)

        pl.loop(0, L)(h_fire)

        def h_drain(l):
            pltpu.make_async_copy(uvt_hbm.at[0].at[nodes_v], hist_t.at[0],
                                  sem_h).wait()

        pl.loop(0, L)(h_drain)

        def h_transpose(l):
            for g in range(8):
                vals = hist_t[l, pl.ds(g * 16, 16)]
                idx = iota200 + (l + g * 16 * L)
                plsc.store_scatter(hist_v, [idx], vals)

        pl.loop(0, L)(h_transpose)
        pltpu.make_async_copy(na_hbm.at[nodes_v], na_v, sem_c).wait()

        for b in range(NBUF):
            fire(b, b)

        def group(g):
            rna16 = _rsqrt16(na_v[pl.ds(g, 16)])
            for j in range(16):
                u = g + j
                slot = j % NBUF
                drain(slot)
                compute(u, slot, rna16[j])
                nxt = u + NBUF

                @pl.when(nxt < C)
                def _():
                    fire(nxt, slot)

        pl.loop(0, C, step=16)(group)
        pltpu.sync_copy(out_v, out_hbm.at[pl.ds(ubase, C)])

    pl.loop(0, NCHUNK)(chunk)


@jax.jit
def _run(nodes, v2e_weight, uv, na_flat, nb_flat):
    mesh = plsc.VectorSubcoreMesh(core_axis_name="c", subcore_axis_name="s")

    v2_lin = pl.kernel(
        _relayout_body,
        out_type=jax.ShapeDtypeStruct((V2_PAD * D,), jnp.float32),
        mesh=mesh,
        scratch_types=[
            pltpu.VMEM((D, 128), jnp.float32),
            pltpu.VMEM((D, 128), jnp.float32),
            pltpu.VMEM((128 * D,), jnp.float32),
            pltpu.VMEM((128 * D,), jnp.float32),
            pltpu.VMEM((128,), jnp.float32),
            pltpu.VMEM((128,), jnp.float32),
        ] + [pltpu.SemaphoreType.DMA] * 6,
        compiler_params=pltpu.CompilerParams(
            needs_layout_passes=False, disable_bounds_checks=True),
    )(v2e_weight.T, nb_flat)


    return pl.kernel(
        _gather_body,
        out_type=jax.ShapeDtypeStruct((B, D), jnp.float32),
        mesh=mesh,
        scratch_types=[
            pltpu.VMEM((C,), jnp.int32),
            pltpu.VMEM((L, C), jnp.int32),
            pltpu.VMEM((C * L,), jnp.int32),
            pltpu.VMEM((C,), jnp.float32),
            pltpu.VMEM((NBUF, LPAD, D), jnp.float32),
            pltpu.VMEM((C, D), jnp.float32),
        ] + [pltpu.SemaphoreType.DMA] * 6,
        compiler_params=pltpu.CompilerParams(
            use_tc_tiling_on_sc=False, needs_layout_passes=False,
            disable_bounds_checks=True),
    )(nodes, v2_lin.reshape(V2_PAD, D), uv.T, na_flat)


def kernel(nodes, v2e_weight, u_v, u_v_l, v_u_l):
    nodes = nodes.astype(jnp.int32)
    uv = u_v.astype(jnp.int32)
    na_flat = u_v_l.reshape(-1).astype(jnp.float32)
    nb_flat = v_u_l.reshape(-1).astype(jnp.float32)
    return _run(nodes, v2e_weight.astype(jnp.float32), uv, na_flat, nb_flat)


# speedup vs baseline: 1.0122x; 1.0122x over previous
import jax
import jax.numpy as jnp
from jax import lax
from jax.experimental import pallas as pl
from jax.experimental.pallas import tpu as pltpu
from jax.experimental.pallas import tpu_sc as plsc

B = 16384
L = 200
D = 16
LPAD = 208
NC, NS = 2, 16
NW = NC * NS
UPW = B // NW
C = 128
NBUF = 4
NCHUNK = UPW // C

N_USERS = 100000
N_ITEMS = 1000000
UB_UV = (N_USERS + 127) // 128
UV_PAD = UB_UV * 128
NB_V2 = (N_ITEMS + 127) // 128
V2_PAD = NB_V2 * 128


def _rsqrt16(x):
    i = plsc.bitcast(x, jnp.int32)
    i = jnp.int32(0x5F3759DF) - lax.shift_right_arithmetic(i, 1)
    y = plsc.bitcast(i, jnp.float32)
    half_x = x * 0.5
    for _ in range(3):
        y = y * (1.5 - half_x * y * y)
    return y


def _relayout_body(v2t_hbm, nb_hbm, v2lin_hbm,
                   tv20, tv21, lv20, lv21, nb0, nb1,
                   srv20, srv21, swv20, swv21, srnb0, srnb1):
    wid = lax.axis_index("c") * NS + lax.axis_index("s")
    iota = lax.iota(jnp.int32, 16)
    iota16 = iota * D

    tv2 = [tv20, tv21]
    lv2 = [lv20, lv21]
    nbv = [nb0, nb1]
    srv2 = [srv20, srv21]
    srnb = [srnb0, srnb1]
    swv2 = [swv20, swv21]
    n_v2 = (NB_V2 - wid + NW - 1) // NW

    def v2_fire_read(seq, b):
        ib = wid + seq * NW
        pltpu.async_copy(v2t_hbm.at[pl.ds(0, D), pl.ds(ib * 128, 128)],
                         tv2[b], srv2[b])
        pltpu.async_copy(nb_hbm.at[pl.ds(ib * 128, 128)], nbv[b], srnb[b])

    def v2_drain_read(b):
        pltpu.make_async_copy(v2t_hbm.at[pl.ds(0, D), pl.ds(0, 128)],
                              tv2[b], srv2[b]).wait()
        pltpu.make_async_copy(nb_hbm.at[pl.ds(0, 128)], nbv[b],
                              srnb[b]).wait()

    def v2_fire_write(seq, b):
        ib = wid + seq * NW
        pltpu.async_copy(lv2[b], v2lin_hbm.at[pl.ds(ib * 128 * D, 128 * D)],
                         swv2[b])

    def v2_drain_write(b):
        pltpu.make_async_copy(lv2[b], v2lin_hbm.at[pl.ds(0, 128 * D)],
                              swv2[b]).wait()

    def v2_process(b):
        for v16 in range(8):
            w16 = _rsqrt16(nbv[b][pl.ds(v16 * 16, 16)])
            for d in range(D):
                vals = tv2[b][d, pl.ds(v16 * 16, 16)] * w16
                idx = iota16 + (d + v16 * 16 * D)
                plsc.store_scatter(lv2[b], [idx], vals)

    v2_fire_read(0, 0)
    v2_fire_read(1, 1)

    def v2_pair(p):
        for b in range(2):
            seq = p * 2 + b

            @pl.when(seq < n_v2)
            def _():
                v2_drain_read(b)

                @pl.when(seq >= 2)
                def _():
                    v2_drain_write(b)

                v2_process(b)
                v2_fire_write(seq, b)

                @pl.when(seq + 2 < n_v2)
                def _():
                    v2_fire_read(seq + 2, b)

    pl.loop(0, (NB_V2 // NW + 2) // 2)(v2_pair)
    v2_drain_write(0)
    v2_drain_write(1)


def _gather_body(nodes_hbm, v2e_hbm, uvt_hbm, na_hbm, out_hbm,
                 nodes_v, hist_t, hist_v, na_v, e_buf, out_v,
                 sem_c, sem_h0, sem_h1, sem_e0, sem_e1, sem_e2, sem_e3):
    sem_e = [sem_e0, sem_e1, sem_e2, sem_e3]
    sem_h = [sem_h0, sem_h1]
    wid = lax.axis_index("c") * NS + lax.axis_index("s")
    base = wid * UPW
    iota = lax.iota(jnp.int32, 16)
    iota200 = iota * L

    zero16 = jnp.zeros((D,), jnp.float32)
    for b in range(NBUF):
        for r in range(L, LPAD):
            e_buf[b, r, :] = zero16

    def fire(u, slot):
        pltpu.async_copy(v2e_hbm.at[hist_v.at[pl.ds(u * L, L)]],
                         e_buf.at[slot, pl.ds(0, L)], sem_e[slot])

    def drain(slot):
        pltpu.make_async_copy(v2e_hbm.at[hist_v.at[pl.ds(0, L)]],
                              e_buf.at[slot, pl.ds(0, L)], sem_e[slot]).wait()

    def compute(u, slot, rna_u):
        def cbody(cth, accs):
            a0, a1, a2, a3 = accs
            r = cth * 16
            for j in range(0, 16, 4):
                a0 = a0 + e_buf[slot, r + j, :]
                a1 = a1 + e_buf[slot, r + j + 1, :]
                a2 = a2 + e_buf[slot, r + j + 2, :]
                a3 = a3 + e_buf[slot, r + j + 3, :]
            return (a0, a1, a2, a3)

        z = jnp.zeros((D,), jnp.float32)
        a0, a1, a2, a3 = lax.fori_loop(0, LPAD // 16, cbody, (z, z, z, z))
        out_v[u, :] = ((a0 + a1) + (a2 + a3)) * rna_u

    def h_fire(ch, hb):
        nref = nodes_v.at[pl.ds(ch * C, C)]

        def per_l(l):
            pltpu.async_copy(uvt_hbm.at[l].at[nref], hist_t.at[hb, l],
                             sem_h[hb])

        pl.loop(0, L)(per_l)

    def h_drain(hb):
        def per_l(l):
            pltpu.make_async_copy(uvt_hbm.at[0].at[nodes_v.at[pl.ds(0, C)]],
                                  hist_t.at[hb, 0], sem_h[hb]).wait()

        pl.loop(0, L)(per_l)

    pltpu.sync_copy(nodes_hbm.at[pl.ds(base, UPW)], nodes_v)
    pltpu.async_copy(na_hbm.at[nodes_v], na_v, sem_c)
    h_fire(0, 0)

    @pl.when(NCHUNK > 1)
    def _():
        h_fire(1, 1)

    pltpu.make_async_copy(na_hbm.at[nodes_v], na_v, sem_c).wait()

    def chunk(ch, hb):
        h_drain(hb)

        def h_transpose(l):
            for g in range(8):
                vals = hist_t[hb, l, pl.ds(g * 16, 16)]
                idx = iota200 + (l + g * 16 * L)
                plsc.store_scatter(hist_v, [idx], vals)

        pl.loop(0, L)(h_transpose)

        @pl.when(ch + 2 < NCHUNK)
        def _():
            h_fire(ch + 2, hb)

        for b in range(NBUF):
            fire(b, b)

        def group(g):
            rna16 = _rsqrt16(na_v[pl.ds(ch * C + g, 16)])
            for j in range(16):
                u = g + j
                slot = j % NBUF
                drain(slot)
                compute(u, slot, rna16[j])
                nxt = u + NBUF

                @pl.when(nxt < C)
                def _():
                    fire(nxt, slot)

        pl.loop(0, C, step=16)(group)
        pltpu.sync_copy(out_v, out_hbm.at[pl.ds(base + ch * C, C)])

    def chunk_pair(p):
        for hb in range(2):
            chunk(p * 2 + hb, hb)

    pl.loop(0, NCHUNK // 2)(chunk_pair)


@jax.jit
def _run(nodes, v2e_weight, uv, na_flat, nb_flat):
    mesh = plsc.VectorSubcoreMesh(core_axis_name="c", subcore_axis_name="s")

    v2_lin = pl.kernel(
        _relayout_body,
        out_type=jax.ShapeDtypeStruct((V2_PAD * D,), jnp.float32),
        mesh=mesh,
        scratch_types=[
            pltpu.VMEM((D, 128), jnp.float32),
            pltpu.VMEM((D, 128), jnp.float32),
            pltpu.VMEM((128 * D,), jnp.float32),
            pltpu.VMEM((128 * D,), jnp.float32),
            pltpu.VMEM((128,), jnp.float32),
            pltpu.VMEM((128,), jnp.float32),
        ] + [pltpu.SemaphoreType.DMA] * 6,
        compiler_params=pltpu.CompilerParams(
            needs_layout_passes=False, disable_bounds_checks=True),
    )(v2e_weight.T, nb_flat)


    return pl.kernel(
        _gather_body,
        out_type=jax.ShapeDtypeStruct((B, D), jnp.float32),
        mesh=mesh,
        scratch_types=[
            pltpu.VMEM((UPW,), jnp.int32),
            pltpu.VMEM((2, L, C), jnp.int32),
            pltpu.VMEM((C * L,), jnp.int32),
            pltpu.VMEM((UPW,), jnp.float32),
            pltpu.VMEM((NBUF, LPAD, D), jnp.float32),
            pltpu.VMEM((C, D), jnp.float32),
        ] + [pltpu.SemaphoreType.DMA] * 7,
        compiler_params=pltpu.CompilerParams(
            use_tc_tiling_on_sc=False, needs_layout_passes=False,
            disable_bounds_checks=True),
    )(nodes, v2_lin.reshape(V2_PAD, D), uv.T, na_flat)


def kernel(nodes, v2e_weight, u_v, u_v_l, v_u_l):
    nodes = nodes.astype(jnp.int32)
    uv = u_v.astype(jnp.int32)
    na_flat = u_v_l.reshape(-1).astype(jnp.float32)
    nb_flat = v_u_l.reshape(-1).astype(jnp.float32)
    return _run(nodes, v2e_weight.astype(jnp.float32), uv, na_flat, nb_flat)
